# baseline (speedup 1.0000x reference)
import functools

import jax
import jax.numpy as jnp
from jax import lax
from jax.experimental import pallas as pl
from jax.experimental.pallas import tpu as pltpu
from jax.experimental.pallas import tpu_sc as plsc

_N = 10000
_E = 320000
_D = 128
_NC = 2
_NS = 16
_NW = _NC * _NS
_EPW = _E // _NW
_K = 80
_GB = _EPW // _K
_KP = 80
_GBP = _EPW // _KP
_NSL = 4
_RPT = 624
_TAIL = _N - _NS * _RPT

_mesh = plsc.VectorSubcoreMesh(core_axis_name="c", subcore_axis_name="s")


def _deg_body(dst_hbm, zeros_hbm, deg0_out, deg1_out, didx_v, ones_v, sem,
              deg_acc):
  c = lax.axis_index("c")
  s = lax.axis_index("s")
  wid = c * _NS + s
  for i in range(_K // 16):
    ones_v[pl.ds(i * 16, 16)] = jnp.ones((16,), jnp.float32)

  @pl.when(s == 0)
  def _():
    pltpu.sync_copy(zeros_hbm, deg_acc)

  pltpu.sync_copy(dst_hbm.at[wid], didx_v)
  plsc.subcore_barrier()

  def body(g, carry):
    pltpu.async_copy(ones_v, deg_acc.at[didx_v.at[g]], sem, add=True)

    @pl.when(g >= 7)
    def _():
      pltpu.make_async_copy(ones_v, deg_acc.at[didx_v.at[0]], sem).wait()

    return carry

  lax.fori_loop(0, _GB, body, 0)

  def drain(g, carry):
    pltpu.make_async_copy(ones_v, deg_acc.at[didx_v.at[0]], sem).wait()
    return carry

  lax.fori_loop(0, 7, drain, 0)
  plsc.subcore_barrier()

  @pl.when(jnp.logical_and(s == 0, c == 0))
  def _():
    pltpu.sync_copy(deg_acc, deg0_out)

  @pl.when(jnp.logical_and(s == 0, c == 1))
  def _():
    pltpu.sync_copy(deg_acc, deg1_out)


_deg_kernel = functools.partial(
    pl.kernel,
    out_type=[jax.ShapeDtypeStruct((_N,), jnp.float32),
              jax.ShapeDtypeStruct((_N,), jnp.float32)],
    mesh=_mesh,
    scratch_types=[
        pltpu.VMEM((_GB, _K), jnp.int32),
        pltpu.VMEM((_K,), jnp.float32),
        pltpu.SemaphoreType.DMA,
        pltpu.VMEM_SHARED((_N,), jnp.float32),
    ],
)(_deg_body)


def _prop_body(y_hbm, src_hbm, dst_hbm, out0_hbm, out1_hbm,
               sidx4, didx4, rows0, rows1, rows2, rows3,
               sg0, sg1, sg2, sg3, ssi0, ssi1, ssi2, ssi3,
               sdi0, sdi1, sdi2, sdi3, acc):
  c = lax.axis_index("c")
  s = lax.axis_index("s")
  wid = c * _NS + s
  rows = (rows0, rows1, rows2, rows3)
  sg = (sg0, sg1, sg2, sg3)
  ssi = (ssi0, ssi1, ssi2, ssi3)
  sdi = (sdi0, sdi1, sdi2, sdi3)

  def zbody(r, carry):
    for cc in range(_D // 16):
      rows0[r, pl.ds(cc * 16, 16)] = jnp.zeros((16,), jnp.float32)
    return carry

  lax.fori_loop(0, _KP, zbody, 0)
  for i in range(_RPT // _KP):
    pltpu.sync_copy(rows0, acc.at[pl.ds(s * _RPT + i * _KP, _KP)])
  pltpu.sync_copy(rows0.at[pl.ds(0, _RPT % _KP)],
                  acc.at[pl.ds(s * _RPT + (_RPT // _KP) * _KP, _RPT % _KP)])

  @pl.when(s == 0)
  def _():
    pltpu.sync_copy(rows0.at[pl.ds(0, _TAIL)],
                    acc.at[pl.ds(_NS * _RPT, _TAIL)])

  plsc.subcore_barrier()

  def _ld_sidx(b, j):
    return pltpu.make_async_copy(src_hbm.at[wid, b], sidx4.at[j], ssi[j])

  def _ld_didx(b, j):
    return pltpu.make_async_copy(dst_hbm.at[wid, b], didx4.at[j], sdi[j])

  def _gather(b, j):
    return pltpu.make_async_copy(y_hbm.at[sidx4.at[j, 0]], rows[j], sg[j])

  for j in range(_NSL):
    pltpu.async_copy(src_hbm.at[wid, j], sidx4.at[j], ssi[j])
    pltpu.async_copy(dst_hbm.at[wid, j], didx4.at[j], sdi[j])
  for j in range(_NSL - 1):
    _ld_sidx(j, j).wait()
    pltpu.async_copy(y_hbm.at[sidx4.at[j, 0]], rows[j], sg[j])

  def body(sb, carry):
    for j in range(_NSL):
      b = _NSL * sb + j
      jj = (j + _NSL - 1) % _NSL

      @pl.when(b + _NSL - 1 < _GBP)
      def _():
        _ld_sidx(b + _NSL - 1, jj).wait()
        pltpu.async_copy(y_hbm.at[sidx4.at[jj, 0]], rows[jj], sg[jj])

      _gather(b, j).wait()
      _ld_didx(b, j).wait()
      pltpu.sync_copy(rows[j], acc.at[didx4.at[j, 0]], add=True)

      @pl.when(b + _NSL < _GBP)
      def _():
        pltpu.async_copy(src_hbm.at[wid, b + _NSL], sidx4.at[j], ssi[j])
        pltpu.async_copy(dst_hbm.at[wid, b + _NSL], didx4.at[j], sdi[j])

    return carry

  lax.fori_loop(0, _GBP // _NSL, body, 0)
  for j in range(_GBP - _NSL * (_GBP // _NSL)):
    b = _NSL * (_GBP // _NSL) + j
    _gather(b, j).wait()
    _ld_didx(b, j).wait()
    pltpu.sync_copy(rows[j], acc.at[didx4.at[j, 0]], add=True)
  plsc.subcore_barrier()

  @pl.when(c == 0)
  def _():
    pltpu.sync_copy(acc.at[pl.ds(s * _RPT, _RPT)],
                    out0_hbm.at[pl.ds(s * _RPT, _RPT)])

  @pl.when(c == 1)
  def _():
    pltpu.sync_copy(acc.at[pl.ds(s * _RPT, _RPT)],
                    out1_hbm.at[pl.ds(s * _RPT, _RPT)])

  @pl.when(jnp.logical_and(s == 0, c == 0))
  def _():
    pltpu.sync_copy(acc.at[pl.ds(_NS * _RPT, _TAIL)],
                    out0_hbm.at[pl.ds(_NS * _RPT, _TAIL)])

  @pl.when(jnp.logical_and(s == 0, c == 1))
  def _():
    pltpu.sync_copy(acc.at[pl.ds(_NS * _RPT, _TAIL)],
                    out1_hbm.at[pl.ds(_NS * _RPT, _TAIL)])


_prop_kernel = functools.partial(
    pl.kernel,
    out_type=[jax.ShapeDtypeStruct((_N, _D), jnp.float32),
              jax.ShapeDtypeStruct((_N, _D), jnp.float32)],
    mesh=_mesh,
    scratch_types=(
        [pltpu.VMEM((_NSL, 1, _KP), jnp.int32),
         pltpu.VMEM((_NSL, 1, _KP), jnp.int32)]
        + [pltpu.VMEM((_KP, _D), jnp.float32)] * _NSL
        + [pltpu.SemaphoreType.DMA] * (3 * _NSL)
        + [pltpu.VMEM_SHARED((_N, _D), jnp.float32)]
    ),
)(_prop_body)


def _enc1_body(degT_ref, x_ref, w1_ref, y1_ref, dinv_ref):
  deg = degT_ref[:, 0:1] + degT_ref[:, 1:2] + 1.0
  dinv = lax.rsqrt(deg)
  xw = jnp.dot(x_ref[...], w1_ref[...], preferred_element_type=jnp.float32)
  y1_ref[...] = dinv * xw
  dinv_ref[...] = dinv


def _mid_body(p0_ref, p1_ref, y1_ref, dinv_ref, b1_ref, w2_ref, y2_ref):
  dinv = dinv_ref[...]
  h = dinv * (p0_ref[...] + p1_ref[...] + y1_ref[...]) + b1_ref[...]
  h = jnp.maximum(h, 0.0)
  y2_ref[...] = dinv * jnp.dot(h, w2_ref[...],
                               preferred_element_type=jnp.float32)


def _dec_body(q0_ref, q1_ref, y2_ref, dinv_ref, b2_ref, wmu_ref, bmu_ref,
              wlv_ref, blv_ref, eps_ref, wd1_ref, bd1_ref, wd2_ref, bd2_ref,
              out_ref):
  h = dinv_ref[...] * (q0_ref[...] + q1_ref[...] + y2_ref[...]) + b2_ref[...]
  h = jnp.maximum(h, 0.0)
  hg = jnp.sum(h, axis=0, keepdims=True) * (1.0 / _N)
  mu = jnp.dot(hg, wmu_ref[...], preferred_element_type=jnp.float32)
  mu = mu + bmu_ref[...]
  lv = jnp.dot(hg, wlv_ref[...], preferred_element_type=jnp.float32)
  lv = lv + blv_ref[...]
  z = mu + jnp.exp(0.5 * lv) * eps_ref[...]
  hd = jnp.dot(z, wd1_ref[...], preferred_element_type=jnp.float32)
  hd = jnp.maximum(hd + bd1_ref[...], 0.0)
  o = jnp.dot(hd, wd2_ref[...], preferred_element_type=jnp.float32)
  out_ref[...] = jnp.tanh(o + bd2_ref[...])


def _tc_call(body, out_shapes):
  return pl.pallas_call(body, out_shape=out_shapes)


def kernel(x, edge_index, batch, eps, W1, b1, W2, b2, Wmu, bmu, Wlv, blv,
           Wd1, bd1, Wd2, bd2):
  del batch
  src4 = edge_index[0].reshape(_NW, _GBP, 1, _KP)
  dst = edge_index[1].reshape(_NW, _GB, _K)
  dst4 = edge_index[1].reshape(_NW, _GBP, 1, _KP)
  zeros1 = jnp.zeros((_N,), jnp.float32)

  deg0, deg1 = _deg_kernel(dst, zeros1)
  degT = jnp.stack([deg0, deg1], axis=1)

  y1, dinv = _tc_call(
      _enc1_body,
      [jax.ShapeDtypeStruct((_N, _D), jnp.float32),
       jax.ShapeDtypeStruct((_N, 1), jnp.float32)],
  )(degT, x, W1)

  p0, p1 = _prop_kernel(y1, src4, dst4)

  y2 = _tc_call(
      _mid_body, jax.ShapeDtypeStruct((_N, _D), jnp.float32)
  )(p0, p1, y1, dinv, b1.reshape(1, _D), W2)

  q0, q1 = _prop_kernel(y2, src4, dst4)

  out = _tc_call(
      _dec_body, jax.ShapeDtypeStruct((1, _N), jnp.float32)
  )(q0, q1, y2, dinv, b2.reshape(1, _D), Wmu, bmu.reshape(1, -1),
    Wlv, blv.reshape(1, -1), eps, Wd1, bd1.reshape(1, -1), Wd2,
    bd2.reshape(1, -1))
  return out


# speedup vs baseline: 1.3110x; 1.3110x over previous
import functools

import jax
import jax.numpy as jnp
from jax import lax
from jax.experimental import pallas as pl
from jax.experimental.pallas import tpu as pltpu
from jax.experimental.pallas import tpu_sc as plsc

_N = 10000
_E = 320000
_D = 128
_NC = 2
_NS = 16
_NW = _NC * _NS
_EPW = _E // _NW
_K = 80
_GB = _EPW // _K
_RPT = 624
_TAIL = _N - _NS * _RPT

_mesh = plsc.VectorSubcoreMesh(core_axis_name="c", subcore_axis_name="s")


def _deg_body(dst_hbm, zeros_hbm, deg0_out, deg1_out, didx_v, ones_v, sem,
              deg_acc):
  c = lax.axis_index("c")
  s = lax.axis_index("s")
  wid = c * _NS + s
  for i in range(_K // 16):
    ones_v[pl.ds(i * 16, 16)] = jnp.ones((16,), jnp.float32)

  @pl.when(s == 0)
  def _():
    pltpu.sync_copy(zeros_hbm, deg_acc)

  pltpu.sync_copy(dst_hbm.at[wid], didx_v)
  plsc.subcore_barrier()

  def body(g, carry):
    pltpu.async_copy(ones_v, deg_acc.at[didx_v.at[g]], sem, add=True)

    @pl.when(g >= 7)
    def _():
      pltpu.make_async_copy(ones_v, deg_acc.at[didx_v.at[0]], sem).wait()

    return carry

  lax.fori_loop(0, _GB, body, 0)

  def drain(g, carry):
    pltpu.make_async_copy(ones_v, deg_acc.at[didx_v.at[0]], sem).wait()
    return carry

  lax.fori_loop(0, 7, drain, 0)
  plsc.subcore_barrier()

  @pl.when(jnp.logical_and(s == 0, c == 0))
  def _():
    pltpu.sync_copy(deg_acc, deg0_out)

  @pl.when(jnp.logical_and(s == 0, c == 1))
  def _():
    pltpu.sync_copy(deg_acc, deg1_out)


_deg_kernel = functools.partial(
    pl.kernel,
    out_type=[jax.ShapeDtypeStruct((_N,), jnp.float32),
              jax.ShapeDtypeStruct((_N,), jnp.float32)],
    mesh=_mesh,
    scratch_types=[
        pltpu.VMEM((_GB, _K), jnp.int32),
        pltpu.VMEM((_K,), jnp.float32),
        pltpu.SemaphoreType.DMA,
        pltpu.VMEM_SHARED((_N,), jnp.float32),
    ],
)(_deg_body)


def _prop_body(y_hbm, src_hbm, dst_hbm, out0_hbm, out1_hbm,
               sidx_v, didx3, rows0, rows1, rows2,
               sg0, sg1, sg2, si0, si1, si2, acc):
  c = lax.axis_index("c")
  s = lax.axis_index("s")
  wid = c * _NS + s
  rows = (rows0, rows1, rows2)
  sg = (sg0, sg1, sg2)
  si = (si0, si1, si2)

  def zbody(r, carry):
    for cc in range(_D // 16):
      rows0[r, pl.ds(cc * 16, 16)] = jnp.zeros((16,), jnp.float32)
    return carry

  lax.fori_loop(0, _K, zbody, 0)
  for i in range(_RPT // _K):
    pltpu.sync_copy(rows0, acc.at[pl.ds(s * _RPT + i * _K, _K)])
  pltpu.sync_copy(rows0.at[pl.ds(0, _RPT % _K)],
                  acc.at[pl.ds(s * _RPT + (_RPT // _K) * _K, _RPT % _K)])

  @pl.when(s == 0)
  def _():
    pltpu.sync_copy(rows0.at[pl.ds(0, _TAIL)],
                    acc.at[pl.ds(_NS * _RPT, _TAIL)])

  pltpu.sync_copy(src_hbm.at[wid], sidx_v)
  plsc.subcore_barrier()

  def _sidx(g):
    return sidx_v.at[pl.ds(g * _K, _K)]

  for j in range(3):
    pltpu.async_copy(dst_hbm.at[wid, j], didx3.at[j], si[j])
    pltpu.async_copy(y_hbm.at[_sidx(j)], rows[j], sg[j])

  def body(sb, carry):
    for j in range(3):
      b = 3 * sb + j
      pltpu.make_async_copy(y_hbm.at[_sidx(b)], rows[j], sg[j]).wait()
      pltpu.make_async_copy(dst_hbm.at[wid, b], didx3.at[j], si[j]).wait()
      pltpu.sync_copy(rows[j], acc.at[didx3.at[j, 0]], add=True)

      @pl.when(b + 3 < _GB)
      def _():
        pltpu.async_copy(dst_hbm.at[wid, b + 3], didx3.at[j], si[j])
        pltpu.async_copy(y_hbm.at[_sidx(b + 3)], rows[j], sg[j])

    return carry

  lax.fori_loop(0, _GB // 3, body, 0)
  for j in range(_GB - 3 * (_GB // 3)):
    b = 3 * (_GB // 3) + j
    pltpu.make_async_copy(y_hbm.at[_sidx(b)], rows[j], sg[j]).wait()
    pltpu.make_async_copy(dst_hbm.at[wid, b], didx3.at[j], si[j]).wait()
    pltpu.sync_copy(rows[j], acc.at[didx3.at[j, 0]], add=True)
  plsc.subcore_barrier()

  @pl.when(c == 0)
  def _():
    pltpu.sync_copy(acc.at[pl.ds(s * _RPT, _RPT)],
                    out0_hbm.at[pl.ds(s * _RPT, _RPT)])

  @pl.when(c == 1)
  def _():
    pltpu.sync_copy(acc.at[pl.ds(s * _RPT, _RPT)],
                    out1_hbm.at[pl.ds(s * _RPT, _RPT)])

  @pl.when(jnp.logical_and(s == 0, c == 0))
  def _():
    pltpu.sync_copy(acc.at[pl.ds(_NS * _RPT, _TAIL)],
                    out0_hbm.at[pl.ds(_NS * _RPT, _TAIL)])

  @pl.when(jnp.logical_and(s == 0, c == 1))
  def _():
    pltpu.sync_copy(acc.at[pl.ds(_NS * _RPT, _TAIL)],
                    out1_hbm.at[pl.ds(_NS * _RPT, _TAIL)])


_prop_kernel = functools.partial(
    pl.kernel,
    out_type=[jax.ShapeDtypeStruct((_N, _D), jnp.float32),
              jax.ShapeDtypeStruct((_N, _D), jnp.float32)],
    mesh=_mesh,
    scratch_types=(
        [pltpu.VMEM((_EPW,), jnp.int32),
         pltpu.VMEM((3, 1, _K), jnp.int32)]
        + [pltpu.VMEM((_K, _D), jnp.float32)] * 3
        + [pltpu.SemaphoreType.DMA] * 6
        + [pltpu.VMEM_SHARED((_N, _D), jnp.float32)]
    ),
)(_prop_body)


def _enc1_body(degT_ref, x_ref, w1_ref, y1_ref, dinv_ref):
  deg = degT_ref[:, 0:1] + degT_ref[:, 1:2] + 1.0
  dinv = lax.rsqrt(deg)
  xw = jnp.dot(x_ref[...], w1_ref[...], preferred_element_type=jnp.float32)
  y1_ref[...] = dinv * xw
  dinv_ref[...] = dinv


def _mid_body(p0_ref, p1_ref, y1_ref, dinv_ref, b1_ref, w2_ref, y2_ref):
  dinv = dinv_ref[...]
  h = dinv * (p0_ref[...] + p1_ref[...] + y1_ref[...]) + b1_ref[...]
  h = jnp.maximum(h, 0.0)
  y2_ref[...] = dinv * jnp.dot(h, w2_ref[...],
                               preferred_element_type=jnp.float32)


def _dec_body(q0_ref, q1_ref, y2_ref, dinv_ref, b2_ref, wmu_ref, bmu_ref,
              wlv_ref, blv_ref, eps_ref, wd1_ref, bd1_ref, wd2_ref, bd2_ref,
              out_ref):
  h = dinv_ref[...] * (q0_ref[...] + q1_ref[...] + y2_ref[...]) + b2_ref[...]
  h = jnp.maximum(h, 0.0)
  hg = jnp.sum(h, axis=0, keepdims=True) * (1.0 / _N)
  mu = jnp.dot(hg, wmu_ref[...], preferred_element_type=jnp.float32)
  mu = mu + bmu_ref[...]
  lv = jnp.dot(hg, wlv_ref[...], preferred_element_type=jnp.float32)
  lv = lv + blv_ref[...]
  z = mu + jnp.exp(0.5 * lv) * eps_ref[...]
  hd = jnp.dot(z, wd1_ref[...], preferred_element_type=jnp.float32)
  hd = jnp.maximum(hd + bd1_ref[...], 0.0)
  o = jnp.dot(hd, wd2_ref[...], preferred_element_type=jnp.float32)
  out_ref[...] = jnp.tanh(o + bd2_ref[...])


def _tc_call(body, out_shapes):
  return pl.pallas_call(body, out_shape=out_shapes)


def kernel(x, edge_index, batch, eps, W1, b1, W2, b2, Wmu, bmu, Wlv, blv,
           Wd1, bd1, Wd2, bd2):
  del batch
  src = edge_index[0].reshape(_NW, _EPW)
  dst = edge_index[1].reshape(_NW, _GB, _K)
  dst4 = edge_index[1].reshape(_NW, _GB, 1, _K)
  zeros1 = jnp.zeros((_N,), jnp.float32)

  deg0, deg1 = _deg_kernel(dst, zeros1)
  degT = jnp.stack([deg0, deg1], axis=1)

  y1, dinv = _tc_call(
      _enc1_body,
      [jax.ShapeDtypeStruct((_N, _D), jnp.float32),
       jax.ShapeDtypeStruct((_N, 1), jnp.float32)],
  )(degT, x, W1)

  p0, p1 = _prop_kernel(y1, src, dst4)

  y2 = _tc_call(
      _mid_body, jax.ShapeDtypeStruct((_N, _D), jnp.float32)
  )(p0, p1, y1, dinv, b1.reshape(1, _D), W2)

  q0, q1 = _prop_kernel(y2, src, dst4)

  out = _tc_call(
      _dec_body, jax.ShapeDtypeStruct((1, _N), jnp.float32)
  )(q0, q1, y2, dinv, b2.reshape(1, _D), Wmu, bmu.reshape(1, -1),
    Wlv, blv.reshape(1, -1), eps, Wd1, bd1.reshape(1, -1), Wd2,
    bd2.reshape(1, -1))
  return out
